# Initial kernel scaffold; baseline (speedup 1.0000x reference)
import jax
import jax.numpy as jnp
from jax.experimental import pallas as pl


def kernel(seq, pos_table):
    raise NotImplementedError("write your pallas kernel here")


# speedup vs baseline: 1.0031x; 1.0031x over previous
import functools

import jax
import jax.numpy as jnp
from jax.experimental import pallas as pl


def _add_body(seq_ref, tab_ref, out_ref):
    out_ref[...] = seq_ref[...] + tab_ref[...][None, :, :]


@functools.partial(jax.jit, static_argnames=())
def _pos_add(seq, pos_table):
    B, S, D = seq.shape
    CHUNK = 128
    grid = (S // CHUNK,)
    return pl.pallas_call(
        _add_body,
        grid=grid,
        in_specs=[
            pl.BlockSpec((B, CHUNK, D), lambda i: (0, i, 0)),
            pl.BlockSpec((CHUNK, D), lambda i: (i, 0)),
        ],
        out_specs=pl.BlockSpec((B, CHUNK, D), lambda i: (0, i, 0)),
        out_shape=jax.ShapeDtypeStruct((B, S, D), seq.dtype),
    )(seq, pos_table)


def kernel(seq, pos_table):
    S = seq.shape[1]
    return _pos_add(seq, pos_table[:S, :])


# speedup vs baseline: 1.0033x; 1.0001x over previous
import functools

import jax
import jax.numpy as jnp
from jax.experimental import pallas as pl
from jax.experimental.pallas import tpu as pltpu


def _add_body(seq_ref, tab_ref, out_ref):
    out_ref[...] = seq_ref[...] + tab_ref[...][None, :, :]


@functools.partial(jax.jit, static_argnames=())
def _pos_add(seq, pos_table):
    B, S, D = seq.shape
    CHUNK = 128
    grid = (S // CHUNK,)
    return pl.pallas_call(
        _add_body,
        grid=grid,
        in_specs=[
            pl.BlockSpec((B, CHUNK, D), lambda i: (0, i, 0)),
            pl.BlockSpec((CHUNK, D), lambda i: (i, 0)),
        ],
        out_specs=pl.BlockSpec((B, CHUNK, D), lambda i: (0, i, 0)),
        out_shape=jax.ShapeDtypeStruct((B, S, D), seq.dtype),
        compiler_params=pltpu.CompilerParams(
            dimension_semantics=("parallel",),
        ),
    )(seq, pos_table)


def kernel(seq, pos_table):
    S = seq.shape[1]
    return _pos_add(seq, pos_table[:S, :])
